# baseline (device time: 44636 ns/iter reference)
import jax
import jax.numpy as jnp
from jax import lax
from jax.experimental import pallas as pl
from jax.experimental.pallas import tpu as pltpu

B, S, H, Dh, Dr = 2, 256, 16, 64, 32
D = 1024
DC_HALF = 64
BS = B * S
F32 = jnp.float32


def kernel(x, Wdkv, Wuk, Wuv, Wq, Wqr, Wkr, Wo):
    def body(x_ref, wdkv_ref, wuk_ref, wuv_ref, wq_ref, wqr_ref, wkr_ref,
             wo_ref, out_ref, c_parts, wuk_parts, wuv_parts, o_acc,
             send_sems, recv_sems):
        peer = (1 - lax.axis_index("x"), lax.axis_index("y"),
                lax.axis_index("z"))

        barrier_sem = pltpu.get_barrier_semaphore()
        pl.semaphore_signal(barrier_sem, inc=1, device_id=peer,
                            device_id_type=pl.DeviceIdType.MESH)
        pl.semaphore_wait(barrier_sem, 1)

        wuk_parts[0, :, :] = wuk_ref[...]
        wuv_parts[0, :, :] = wuv_ref[...]
        x2d = x_ref[...].reshape(BS, D)
        c_parts[0, :, :] = jnp.dot(x2d, wdkv_ref[...],
                                   preferred_element_type=F32)

        rdmas = []
        for i, buf in enumerate((c_parts, wuk_parts, wuv_parts)):
            rdma = pltpu.make_async_remote_copy(
                src_ref=buf.at[0],
                dst_ref=buf.at[1],
                send_sem=send_sems.at[i],
                recv_sem=recv_sems.at[i],
                device_id=peer,
                device_id_type=pl.DeviceIdType.MESH,
            )
            rdma.start()
            rdmas.append(rdma)

        q2d = jnp.dot(x2d, wq_ref[...], preferred_element_type=F32)
        qr2d = jnp.dot(x2d, wqr_ref[...], preferred_element_type=F32)
        kr2d = jnp.dot(x2d, wkr_ref[...], preferred_element_type=F32)

        for rdma in rdmas:
            rdma.wait()

        k2d = (jnp.dot(c_parts[0, :, :], wuk_parts[0, :, :],
                       preferred_element_type=F32)
               + jnp.dot(c_parts[1, :, :], wuk_parts[1, :, :],
                         preferred_element_type=F32))
        v2d = (jnp.dot(c_parts[0, :, :], wuv_parts[0, :, :],
                       preferred_element_type=F32)
               + jnp.dot(c_parts[1, :, :], wuv_parts[1, :, :],
                         preferred_element_type=F32))

        scale = (Dh + Dr) ** -0.5
        trans = (((1,), (1,)), ((), ()))
        for b in range(B):
            rows = slice(b * S, (b + 1) * S)
            kr_b = kr2d[rows, :]
            for h in range(H):
                cols = slice(h * Dh, (h + 1) * Dh)
                q = q2d[rows, cols]
                qr = qr2d[rows, h * Dr:(h + 1) * Dr]
                k = k2d[rows, cols]
                v = v2d[rows, cols]
                s = (lax.dot_general(q, k, trans, preferred_element_type=F32)
                     + lax.dot_general(qr, kr_b, trans,
                                       preferred_element_type=F32)) * scale
                m = jnp.max(s, axis=1, keepdims=True)
                p = jnp.exp(s - m)
                p = p / jnp.sum(p, axis=1, keepdims=True)
                o_acc[rows, cols] = jnp.dot(p, v, preferred_element_type=F32)

        out2d = jnp.dot(o_acc[...], wo_ref[...], preferred_element_type=F32)
        out_ref[...] = out2d.reshape(B, S, D)

    return pl.pallas_call(
        body,
        out_shape=jax.ShapeDtypeStruct((B, S, D), F32),
        in_specs=[pl.BlockSpec(memory_space=pltpu.VMEM)] * 8,
        out_specs=pl.BlockSpec(memory_space=pltpu.VMEM),
        scratch_shapes=[
            pltpu.VMEM((2, BS, DC_HALF), F32),
            pltpu.VMEM((2, DC_HALF, D), F32),
            pltpu.VMEM((2, DC_HALF, D), F32),
            pltpu.VMEM((BS, H * Dh), F32),
            pltpu.SemaphoreType.DMA((3,)),
            pltpu.SemaphoreType.DMA((3,)),
        ],
        compiler_params=pltpu.CompilerParams(collective_id=0),
    )(x, Wdkv, Wuk, Wuv, Wq, Wqr, Wkr, Wo)


# device time: 43261 ns/iter; 1.0318x vs baseline; 1.0318x over previous
import jax
import jax.numpy as jnp
from jax import lax
from jax.experimental import pallas as pl
from jax.experimental.pallas import tpu as pltpu

B, S, H, Dh, Dr = 2, 256, 16, 64, 32
D = 1024
DC_HALF = 64
SQ = S // 2
F32 = jnp.float32


def kernel(x, Wdkv, Wuk, Wuv, Wq, Wqr, Wkr, Wo):
    def body(x_ref, wdkv_ref, wuk_ref, wuv_ref, wq_ref, wqr_ref, wkr_ref,
             wo_ref, out_ref, c_parts, wuk_parts, wuv_parts, out_parts,
             send_sems, recv_sems):
        my_x = lax.axis_index("x")
        my_y = lax.axis_index("y")
        my_z = lax.axis_index("z")
        x_peer = (1 - my_x, my_y, my_z)
        z_peer = (my_x, my_y, 1 - my_z)
        y_peer = (my_x, 1 - my_y, my_z)
        yz_peer = (my_x, 1 - my_y, 1 - my_z)

        barrier_sem = pltpu.get_barrier_semaphore()
        for p in (x_peer, z_peer, y_peer, yz_peer):
            pl.semaphore_signal(barrier_sem, inc=1, device_id=p,
                                device_id_type=pl.DeviceIdType.MESH)
        pl.semaphore_wait(barrier_sem, 4)

        xb = x_ref[pl.ds(my_y, 1), :, :].reshape(S, D)
        wuk_parts[0, :, :] = wuk_ref[...]
        wuv_parts[0, :, :] = wuv_ref[...]
        c_parts[0, :, :] = jnp.dot(xb, wdkv_ref[...],
                                   preferred_element_type=F32)

        x_rdmas = []
        for i, buf in enumerate((c_parts, wuk_parts, wuv_parts)):
            rdma = pltpu.make_async_remote_copy(
                src_ref=buf.at[0],
                dst_ref=buf.at[1],
                send_sem=send_sems.at[i],
                recv_sem=recv_sems.at[i],
                device_id=x_peer,
                device_id_type=pl.DeviceIdType.MESH,
            )
            rdma.start()
            x_rdmas.append(rdma)

        xq = x_ref[pl.ds(my_y, 1), pl.ds(my_z * SQ, SQ), :].reshape(SQ, D)
        q2d = jnp.dot(xq, wq_ref[...], preferred_element_type=F32)
        qr2d = jnp.dot(xq, wqr_ref[...], preferred_element_type=F32)
        kr_b = jnp.dot(xb, wkr_ref[...], preferred_element_type=F32)

        for rdma in x_rdmas:
            rdma.wait()

        k2d = (jnp.dot(c_parts[0, :, :], wuk_parts[0, :, :],
                       preferred_element_type=F32)
               + jnp.dot(c_parts[1, :, :], wuk_parts[1, :, :],
                         preferred_element_type=F32))
        v2d = (jnp.dot(c_parts[0, :, :], wuv_parts[0, :, :],
                       preferred_element_type=F32)
               + jnp.dot(c_parts[1, :, :], wuv_parts[1, :, :],
                         preferred_element_type=F32))

        scale = (Dh + Dr) ** -0.5
        trans = (((1,), (1,)), ((), ()))
        o_cols = []
        for h in range(H):
            cols = slice(h * Dh, (h + 1) * Dh)
            qcat = jnp.concatenate(
                [q2d[:, cols], qr2d[:, h * Dr:(h + 1) * Dr]], axis=1)
            kcat = jnp.concatenate([k2d[:, cols], kr_b], axis=1)
            s = lax.dot_general(qcat, kcat, trans,
                                preferred_element_type=F32) * scale
            m = jnp.max(s, axis=1, keepdims=True)
            e = jnp.exp(s - m)
            o = jnp.dot(e, v2d[:, cols], preferred_element_type=F32)
            o_cols.append(o / jnp.sum(e, axis=1, keepdims=True))
        o2d = jnp.concatenate(o_cols, axis=1)

        out_parts[0, :, :] = jnp.dot(o2d, wo_ref[...],
                                     preferred_element_type=F32)

        out_rdmas = []
        for i, p in enumerate((z_peer, y_peer, yz_peer)):
            rdma = pltpu.make_async_remote_copy(
                src_ref=out_parts.at[0],
                dst_ref=out_parts.at[i + 1],
                send_sem=send_sems.at[3 + i],
                recv_sem=recv_sems.at[3 + i],
                device_id=p,
                device_id_type=pl.DeviceIdType.MESH,
            )
            rdma.start()
            out_rdmas.append(rdma)

        out_ref[pl.ds(my_y, 1), pl.ds(my_z * SQ, SQ), :] = (
            out_parts[0, :, :].reshape(1, SQ, D))

        places = [
            (my_y, (1 - my_z) * SQ),
            (1 - my_y, my_z * SQ),
            (1 - my_y, (1 - my_z) * SQ),
        ]
        for i, (rdma, (yy, r0)) in enumerate(zip(out_rdmas, places)):
            rdma.wait()
            out_ref[pl.ds(yy, 1), pl.ds(r0, SQ), :] = (
                out_parts[i + 1, :, :].reshape(1, SQ, D))

    return pl.pallas_call(
        body,
        out_shape=jax.ShapeDtypeStruct((B, S, D), F32),
        in_specs=[pl.BlockSpec(memory_space=pltpu.VMEM)] * 8,
        out_specs=pl.BlockSpec(memory_space=pltpu.VMEM),
        scratch_shapes=[
            pltpu.VMEM((2, S, DC_HALF), F32),
            pltpu.VMEM((2, DC_HALF, D), F32),
            pltpu.VMEM((2, DC_HALF, D), F32),
            pltpu.VMEM((4, SQ, D), F32),
            pltpu.SemaphoreType.DMA((6,)),
            pltpu.SemaphoreType.DMA((6,)),
        ],
        compiler_params=pltpu.CompilerParams(collective_id=0),
    )(x, Wdkv, Wuk, Wuv, Wq, Wqr, Wkr, Wo)
